# initial kernel scaffold (unmeasured)
import jax
import jax.numpy as jnp
from jax import lax
from jax.experimental import pallas as pl
from jax.experimental.pallas import tpu as pltpu

NZ = 4


def _ring_reduce_scatter(Pp):
    _, B, S_per, N = Pp.shape

    def body(p_ref, out_ref, recv_ref, acc_ref, va, vb, vo, vof,
             lsems, ssems, rsems):
        x = lax.axis_index("x")
        y = lax.axis_index("y")
        z = lax.axis_index("z")
        left = (z - 1) % NZ
        right = (z + 1) % NZ

        barrier = pltpu.get_barrier_semaphore()
        for nbr in (left, right):
            pl.semaphore_signal(
                barrier, inc=1,
                device_id=(x, y, nbr), device_id_type=pl.DeviceIdType.MESH,
            )
        pl.semaphore_wait(barrier, 2)

        def add_pass(t, dst_ref, final):
            for b in range(B):
                ca = pltpu.make_async_copy(recv_ref.at[t, b], va, lsems.at[0])
                cb = pltpu.make_async_copy(p_ref.at[t + 1, b], vb, lsems.at[1])
                ca.start()
                cb.start()
                ca.wait()
                cb.wait()
                if final:
                    vof[...] = va[...].astype(jnp.float32) + vb[...].astype(jnp.float32)
                    co = pltpu.make_async_copy(vof, dst_ref.at[b], lsems.at[2])
                else:
                    vo[...] = va[...] + vb[...]
                    co = pltpu.make_async_copy(vo, dst_ref.at[b], lsems.at[2])
                co.start()
                co.wait()

        for t in range(NZ - 1):
            src = p_ref.at[0] if t == 0 else acc_ref
            rdma = pltpu.make_async_remote_copy(
                src_ref=src,
                dst_ref=recv_ref.at[t],
                send_sem=ssems.at[t],
                recv_sem=rsems.at[t],
                device_id=(x, y, right),
                device_id_type=pl.DeviceIdType.MESH,
            )
            rdma.start()
            rdma.wait()
            if t < NZ - 2:
                add_pass(t, acc_ref, final=False)
            else:
                add_pass(t, out_ref, final=True)

    return pl.pallas_call(
        body,
        out_shape=jax.ShapeDtypeStruct((B, S_per, N), jnp.float32),
        in_specs=[pl.BlockSpec(memory_space=pltpu.ANY)],
        out_specs=pl.BlockSpec(memory_space=pltpu.ANY),
        scratch_shapes=[
            pltpu.ANY((NZ - 1, B, S_per, N), jnp.bfloat16),
            pltpu.ANY((B, S_per, N), jnp.bfloat16),
            pltpu.VMEM((S_per, N), jnp.bfloat16),
            pltpu.VMEM((S_per, N), jnp.bfloat16),
            pltpu.VMEM((S_per, N), jnp.bfloat16),
            pltpu.VMEM((S_per, N), jnp.float32),
            pltpu.SemaphoreType.DMA((3,)),
            pltpu.SemaphoreType.DMA((NZ - 1,)),
            pltpu.SemaphoreType.DMA((NZ - 1,)),
        ],
        compiler_params=pltpu.CompilerParams(collective_id=0),
    )(Pp)


def kernel(O, Wo):
    B, S, Hp, D = O.shape
    S_per = S // NZ
    z = lax.axis_index("z")

    Ob = O.reshape(B, S, Hp * D).astype(jnp.bfloat16)
    Wb = Wo.astype(jnp.bfloat16)

    chunks = []
    for j in range(NZ):
        c = (z - 1 - j) % NZ
        Oc = lax.dynamic_slice_in_dim(Ob, c * S_per, S_per, axis=1)
        P = jnp.matmul(Oc, Wb, preferred_element_type=jnp.float32)
        chunks.append(P.astype(jnp.bfloat16))
    Pp = jnp.stack(chunks, axis=0)

    return _ring_reduce_scatter(Pp)


# baseline (device time: 1793930 ns/iter reference)
import jax
import jax.numpy as jnp
from jax import lax
from jax.experimental import pallas as pl
from jax.experimental.pallas import tpu as pltpu

NZ = 4


def _ring_reduce_scatter(Pp):
    _, B, S_per, N = Pp.shape

    def body(p_ref, out_ref, recv_ref, acc_ref, va, vb, vof,
             lsems, ssems, rsems):
        x = lax.axis_index("x")
        y = lax.axis_index("y")
        z = lax.axis_index("z")
        left = (z - 1) % NZ
        right = (z + 1) % NZ

        barrier = pltpu.get_barrier_semaphore()
        for nbr in (left, right):
            pl.semaphore_signal(
                barrier, inc=1,
                device_id=(x, y, nbr), device_id_type=pl.DeviceIdType.MESH,
            )
        pl.semaphore_wait(barrier, 2)

        def add_pass(t, dst_ref, final):
            for b in range(B):
                ca = pltpu.make_async_copy(recv_ref.at[t, b], va, lsems.at[0])
                cb = pltpu.make_async_copy(p_ref.at[t + 1, b], vb, lsems.at[1])
                ca.start()
                cb.start()
                ca.wait()
                cb.wait()
                if final:
                    vof[...] = va[...].astype(jnp.float32) + vb[...].astype(jnp.float32)
                    co = pltpu.make_async_copy(vof, dst_ref.at[b], lsems.at[2])
                else:
                    va[...] = va[...] + vb[...]
                    co = pltpu.make_async_copy(va, dst_ref.at[b], lsems.at[2])
                co.start()
                co.wait()

        for t in range(NZ - 1):
            src = p_ref.at[0] if t == 0 else acc_ref
            rdma = pltpu.make_async_remote_copy(
                src_ref=src,
                dst_ref=recv_ref.at[t],
                send_sem=ssems.at[t],
                recv_sem=rsems.at[t],
                device_id=(x, y, right),
                device_id_type=pl.DeviceIdType.MESH,
            )
            rdma.start()
            rdma.wait()
            if t < NZ - 2:
                add_pass(t, acc_ref, final=False)
            else:
                add_pass(t, out_ref, final=True)

    out, _, _ = pl.pallas_call(
        body,
        out_shape=[
            jax.ShapeDtypeStruct((B, S_per, N), jnp.float32),
            jax.ShapeDtypeStruct((NZ - 1, B, S_per, N), jnp.bfloat16),
            jax.ShapeDtypeStruct((B, S_per, N), jnp.bfloat16),
        ],
        in_specs=[pl.BlockSpec(memory_space=pl.ANY)],
        out_specs=[
            pl.BlockSpec(memory_space=pl.ANY),
            pl.BlockSpec(memory_space=pl.ANY),
            pl.BlockSpec(memory_space=pl.ANY),
        ],
        scratch_shapes=[
            pltpu.VMEM((S_per, N), jnp.bfloat16),
            pltpu.VMEM((S_per, N), jnp.bfloat16),
            pltpu.VMEM((S_per, N), jnp.float32),
            pltpu.SemaphoreType.DMA((3,)),
            pltpu.SemaphoreType.DMA((NZ - 1,)),
            pltpu.SemaphoreType.DMA((NZ - 1,)),
        ],
        compiler_params=pltpu.CompilerParams(
            collective_id=0, vmem_limit_bytes=100 * 1024 * 1024,
        ),
    )(Pp)
    return out


def kernel(O, Wo):
    B, S, Hp, D = O.shape
    S_per = S // NZ
    z = lax.axis_index("z")

    Ob = O.reshape(B, S, Hp * D).astype(jnp.bfloat16)
    Wb = Wo.astype(jnp.bfloat16)

    chunks = []
    for j in range(NZ):
        c = (z - 1 - j) % NZ
        Oc = lax.dynamic_slice_in_dim(Ob, c * S_per, S_per, axis=1)
        P = jnp.matmul(Oc, Wb, preferred_element_type=jnp.float32)
        chunks.append(P.astype(jnp.bfloat16))
    Pp = jnp.stack(chunks, axis=0)

    return _ring_reduce_scatter(Pp)


# device time: 1672776 ns/iter; 1.0724x vs baseline; 1.0724x over previous
import jax
import jax.numpy as jnp
from jax import lax
from jax.experimental import pallas as pl
from jax.experimental.pallas import tpu as pltpu

NZ = 4


def _ring_reduce_scatter(Pp):
    _, B, S_per, N = Pp.shape

    def body(p_ref, out_ref, recv_ref, acc_ref, va, vb, vof,
             lsems, ssems, rsems):
        x = lax.axis_index("x")
        y = lax.axis_index("y")
        z = lax.axis_index("z")
        left = (z - 1) % NZ
        right = (z + 1) % NZ

        barrier = pltpu.get_barrier_semaphore()
        for nbr in (left, right):
            pl.semaphore_signal(
                barrier, inc=1,
                device_id=(x, y, nbr), device_id_type=pl.DeviceIdType.MESH,
            )
        pl.semaphore_wait(barrier, 2)

        sends = []

        def _desc(t, b, src):
            return pltpu.make_async_remote_copy(
                src_ref=src,
                dst_ref=recv_ref.at[t, b],
                send_sem=ssems.at[t, b],
                recv_sem=rsems.at[t, b],
                device_id=(x, y, right),
                device_id_type=pl.DeviceIdType.MESH,
            )

        def start_send(t, b, src):
            r = _desc(t, b, src)
            r.start()
            sends.append(r)

        def wait_recv(t, b):
            _desc(t, b, p_ref.at[0, b]).wait_recv()

        for b in range(B):
            start_send(0, b, p_ref.at[0, b])
        for t in range(NZ - 2):
            for b in range(B):
                wait_recv(t, b)
                ca = pltpu.make_async_copy(recv_ref.at[t, b], va, lsems.at[0])
                cb = pltpu.make_async_copy(p_ref.at[t + 1, b], vb, lsems.at[1])
                ca.start()
                cb.start()
                ca.wait()
                cb.wait()
                va[...] = va[...] + vb[...]
                co = pltpu.make_async_copy(va, acc_ref.at[t, b], lsems.at[2])
                co.start()
                co.wait()
                start_send(t + 1, b, acc_ref.at[t, b])
        for b in range(B):
            t = NZ - 2
            wait_recv(t, b)
            ca = pltpu.make_async_copy(recv_ref.at[t, b], va, lsems.at[0])
            cb = pltpu.make_async_copy(p_ref.at[t + 1, b], vb, lsems.at[1])
            ca.start()
            cb.start()
            ca.wait()
            cb.wait()
            vof[...] = va[...].astype(jnp.float32) + vb[...].astype(jnp.float32)
            co = pltpu.make_async_copy(vof, out_ref.at[b], lsems.at[2])
            co.start()
            co.wait()
        for r in sends:
            r.wait_send()

    out, _, _ = pl.pallas_call(
        body,
        out_shape=[
            jax.ShapeDtypeStruct((B, S_per, N), jnp.float32),
            jax.ShapeDtypeStruct((NZ - 1, B, S_per, N), jnp.bfloat16),
            jax.ShapeDtypeStruct((NZ - 2, B, S_per, N), jnp.bfloat16),
        ],
        in_specs=[pl.BlockSpec(memory_space=pl.ANY)],
        out_specs=[
            pl.BlockSpec(memory_space=pl.ANY),
            pl.BlockSpec(memory_space=pl.ANY),
            pl.BlockSpec(memory_space=pl.ANY),
        ],
        scratch_shapes=[
            pltpu.VMEM((S_per, N), jnp.bfloat16),
            pltpu.VMEM((S_per, N), jnp.bfloat16),
            pltpu.VMEM((S_per, N), jnp.float32),
            pltpu.SemaphoreType.DMA((3,)),
            pltpu.SemaphoreType.DMA((NZ - 1, 4)),
            pltpu.SemaphoreType.DMA((NZ - 1, 4)),
        ],
        compiler_params=pltpu.CompilerParams(
            collective_id=0, vmem_limit_bytes=100 * 1024 * 1024,
        ),
    )(Pp)
    return out


def kernel(O, Wo):
    B, S, Hp, D = O.shape
    S_per = S // NZ
    z = lax.axis_index("z")

    Ob = O.reshape(B, S, Hp * D).astype(jnp.bfloat16)
    Wb = Wo.astype(jnp.bfloat16)

    chunks = []
    for j in range(NZ):
        c = (z - 1 - j) % NZ
        Oc = lax.dynamic_slice_in_dim(Ob, c * S_per, S_per, axis=1)
        P = jnp.matmul(Oc, Wb, preferred_element_type=jnp.float32)
        chunks.append(P.astype(jnp.bfloat16))
    Pp = jnp.stack(chunks, axis=0)

    return _ring_reduce_scatter(Pp)


# device time: 1299372 ns/iter; 1.3806x vs baseline; 1.2874x over previous
import jax

jax.config.update("jax_compilation_cache_dir", "/tmp/scband_jax_cache")
jax.config.update("jax_persistent_cache_min_compile_time_secs", 1.0)

import jax.numpy as jnp
from jax import lax
from jax.experimental import pallas as pl
from jax.experimental.pallas import tpu as pltpu

NZ = 4


def _fused_mm_reduce_scatter(Ob, Wb):
    B, S, K = Ob.shape
    N = Wb.shape[1]
    S_per = S // NZ
    NH = 2
    Nh = N // NH

    def body(ob_ref, wb_ref, out_ref, pown0_ref, recv_ref, acc_ref,
             wvm, ovm, va, vacc, osems, lsems, ssems, rsems):
        x = lax.axis_index("x")
        y = lax.axis_index("y")
        z = lax.axis_index("z")
        left = (z - 1) % NZ
        right = (z + 1) % NZ

        wload = pltpu.make_async_copy(wb_ref, wvm, lsems.at[0])
        wload.start()

        barrier = pltpu.get_barrier_semaphore()
        for nbr in (left, right):
            pl.semaphore_signal(
                barrier, inc=1,
                device_id=(x, y, nbr), device_id_type=pl.DeviceIdType.MESH,
            )
        pl.semaphore_wait(barrier, 2)

        wload.wait()

        def _desc(t, b, src):
            return pltpu.make_async_remote_copy(
                src_ref=src,
                dst_ref=recv_ref.at[t, b],
                send_sem=ssems.at[t, b],
                recv_sem=rsems.at[t, b],
                device_id=(x, y, right),
                device_id_type=pl.DeviceIdType.MESH,
            )

        def load_o(j, b):
            c = (z - 1 - j) % NZ
            cp = pltpu.make_async_copy(
                ob_ref.at[b, pl.ds(c * S_per, S_per), :], ovm, osems.at[0]
            )
            cp.start()
            cp.wait()

        for j in range(NZ):
            t = j - 1

            def b_body(b, carry, j=j, t=t):
                load_o(j, b)
                if j > 0:
                    _desc(t, b, pown0_ref.at[b]).wait_recv()
                for h in range(NH):
                    m32 = jnp.dot(
                        ovm[...],
                        wvm[:, h * Nh:(h + 1) * Nh],
                        preferred_element_type=jnp.float32,
                    )
                    if j == 0:
                        vacc[...] = m32.astype(jnp.bfloat16)
                        dst = pown0_ref.at[b, :, pl.ds(h * Nh, Nh)]
                    else:
                        ca = pltpu.make_async_copy(
                            recv_ref.at[t, b, slice(None), pl.ds(h * Nh, Nh)],
                            va, lsems.at[1],
                        )
                        ca.start()
                        ca.wait()
                        vacc[...] = (
                            va[...].astype(jnp.float32) + m32
                        ).astype(jnp.bfloat16)
                        if j < NZ - 1:
                            dst = acc_ref.at[t, b, slice(None), pl.ds(h * Nh, Nh)]
                        else:
                            dst = out_ref.at[b, :, pl.ds(h * Nh, Nh)]
                    co = pltpu.make_async_copy(vacc, dst, lsems.at[2])
                    co.start()
                    co.wait()
                if j == 0:
                    _desc(0, b, pown0_ref.at[b]).start()
                elif j < NZ - 1:
                    _desc(j, b, acc_ref.at[t, b]).start()
                return carry

            lax.fori_loop(0, B, b_body, 0)

        for t in range(NZ - 1):
            def ws_body(b, carry, t=t):
                src = pown0_ref.at[b] if t == 0 else acc_ref.at[t - 1, b]
                _desc(t, b, src).wait_send()
                return carry

            lax.fori_loop(0, B, ws_body, 0)

    out, _, _, _ = pl.pallas_call(
        body,
        out_shape=[
            jax.ShapeDtypeStruct((B, S_per, N), jnp.bfloat16),
            jax.ShapeDtypeStruct((B, S_per, N), jnp.bfloat16),
            jax.ShapeDtypeStruct((NZ - 1, B, S_per, N), jnp.bfloat16),
            jax.ShapeDtypeStruct((NZ - 2, B, S_per, N), jnp.bfloat16),
        ],
        in_specs=[
            pl.BlockSpec(memory_space=pl.ANY),
            pl.BlockSpec(memory_space=pl.ANY),
        ],
        out_specs=[
            pl.BlockSpec(memory_space=pl.ANY),
            pl.BlockSpec(memory_space=pl.ANY),
            pl.BlockSpec(memory_space=pl.ANY),
            pl.BlockSpec(memory_space=pl.ANY),
        ],
        scratch_shapes=[
            pltpu.VMEM((K, N), jnp.bfloat16),
            pltpu.VMEM((S_per, K), jnp.bfloat16),
            pltpu.VMEM((S_per, Nh), jnp.bfloat16),
            pltpu.VMEM((S_per, Nh), jnp.bfloat16),
            pltpu.SemaphoreType.DMA((1,)),
            pltpu.SemaphoreType.DMA((3,)),
            pltpu.SemaphoreType.DMA((NZ - 1, 4)),
            pltpu.SemaphoreType.DMA((NZ - 1, 4)),
        ],
        compiler_params=pltpu.CompilerParams(
            collective_id=0, vmem_limit_bytes=62 * 1024 * 1024,
        ),
    )(Ob, Wb)
    return out.astype(jnp.float32)


def kernel(O, Wo):
    B, S, Hp, D = O.shape
    Ob = O.reshape(B, S, Hp * D).astype(jnp.bfloat16)
    Wb = Wo.astype(jnp.bfloat16)
    return _fused_mm_reduce_scatter(Ob, Wb)


# device time: 1291421 ns/iter; 1.3891x vs baseline; 1.0062x over previous
import jax

jax.config.update("jax_compilation_cache_dir", "/tmp/scband_jax_cache")
jax.config.update("jax_persistent_cache_min_compile_time_secs", 1.0)

import jax.numpy as jnp
from jax import lax
from jax.experimental import pallas as pl
from jax.experimental.pallas import tpu as pltpu

NZ = 4


def _fused_mm_reduce_scatter(Ob, Wb):
    B, S, K = Ob.shape
    N = Wb.shape[1]
    S_per = S // NZ
    NH = 2
    Nh = N // NH

    def body(ob_ref, wb_ref, out_ref, pown0_ref, recv_ref, acc_ref,
             wvm, ovm, va, vacc, osems, lsems, stsems, ssems, rsems):
        x = lax.axis_index("x")
        y = lax.axis_index("y")
        z = lax.axis_index("z")
        left = (z - 1) % NZ
        right = (z + 1) % NZ

        wload = pltpu.make_async_copy(wb_ref, wvm, lsems.at[0])
        wload.start()

        barrier = pltpu.get_barrier_semaphore()
        for nbr in (left, right):
            pl.semaphore_signal(
                barrier, inc=1,
                device_id=(x, y, nbr), device_id_type=pl.DeviceIdType.MESH,
            )
        pl.semaphore_wait(barrier, 2)

        wload.wait()

        def _desc(t, b, src):
            return pltpu.make_async_remote_copy(
                src_ref=src,
                dst_ref=recv_ref.at[t, b],
                send_sem=ssems.at[t, b],
                recv_sem=rsems.at[t, b],
                device_id=(x, y, right),
                device_id_type=pl.DeviceIdType.MESH,
            )

        for j in range(NZ):
            t = j - 1

            def b_body(b, carry, j=j, t=t):
                c = (z - 1 - j) % NZ
                ol = pltpu.make_async_copy(
                    ob_ref.at[b, pl.ds(c * S_per, S_per), :], ovm, osems.at[0]
                )
                ol.start()
                if j > 0:
                    _desc(t, b, pown0_ref.at[b]).wait_recv()
                    ca = pltpu.make_async_copy(
                        recv_ref.at[t, b], va, lsems.at[1]
                    )
                    ca.start()
                ol.wait()
                stores = []
                for h in range(NH):
                    m32 = jnp.dot(
                        ovm[...],
                        wvm[:, h * Nh:(h + 1) * Nh],
                        preferred_element_type=jnp.float32,
                    )
                    if j == 0:
                        vacc[h] = m32.astype(jnp.bfloat16)
                        dst = pown0_ref.at[b, :, pl.ds(h * Nh, Nh)]
                    else:
                        if h == 0:
                            ca.wait()
                        vacc[h] = (
                            va[:, h * Nh:(h + 1) * Nh].astype(jnp.float32)
                            + m32
                        ).astype(jnp.bfloat16)
                        if j < NZ - 1:
                            dst = acc_ref.at[t, b, slice(None), pl.ds(h * Nh, Nh)]
                        else:
                            dst = out_ref.at[b, :, pl.ds(h * Nh, Nh)]
                    co = pltpu.make_async_copy(vacc.at[h], dst, stsems.at[h])
                    co.start()
                    stores.append(co)
                for co in stores:
                    co.wait()
                if j == 0:
                    _desc(0, b, pown0_ref.at[b]).start()
                elif j < NZ - 1:
                    _desc(j, b, acc_ref.at[t, b]).start()
                return carry

            lax.fori_loop(0, B, b_body, 0)

        for t in range(NZ - 1):
            def ws_body(b, carry, t=t):
                src = pown0_ref.at[b] if t == 0 else acc_ref.at[t - 1, b]
                _desc(t, b, src).wait_send()
                return carry

            lax.fori_loop(0, B, ws_body, 0)

    out, _, _, _ = pl.pallas_call(
        body,
        out_shape=[
            jax.ShapeDtypeStruct((B, S_per, N), jnp.bfloat16),
            jax.ShapeDtypeStruct((B, S_per, N), jnp.bfloat16),
            jax.ShapeDtypeStruct((NZ - 1, B, S_per, N), jnp.bfloat16),
            jax.ShapeDtypeStruct((NZ - 2, B, S_per, N), jnp.bfloat16),
        ],
        in_specs=[
            pl.BlockSpec(memory_space=pl.ANY),
            pl.BlockSpec(memory_space=pl.ANY),
        ],
        out_specs=[
            pl.BlockSpec(memory_space=pl.ANY),
            pl.BlockSpec(memory_space=pl.ANY),
            pl.BlockSpec(memory_space=pl.ANY),
            pl.BlockSpec(memory_space=pl.ANY),
        ],
        scratch_shapes=[
            pltpu.VMEM((K, N), jnp.bfloat16),
            pltpu.VMEM((S_per, K), jnp.bfloat16),
            pltpu.VMEM((S_per, N), jnp.bfloat16),
            pltpu.VMEM((NH, S_per, Nh), jnp.bfloat16),
            pltpu.SemaphoreType.DMA((1,)),
            pltpu.SemaphoreType.DMA((2,)),
            pltpu.SemaphoreType.DMA((NH,)),
            pltpu.SemaphoreType.DMA((NZ - 1, 4)),
            pltpu.SemaphoreType.DMA((NZ - 1, 4)),
        ],
        compiler_params=pltpu.CompilerParams(
            collective_id=0, vmem_limit_bytes=64 * 1024 * 1024,
        ),
    )(Ob, Wb)
    return out.astype(jnp.float32)


def kernel(O, Wo):
    B, S, Hp, D = O.shape
    Ob = O.reshape(B, S, Hp * D).astype(jnp.bfloat16)
    Wb = Wo.astype(jnp.bfloat16)
    return _fused_mm_reduce_scatter(Ob, Wb)
